# baseline (speedup 1.0000x reference)
import functools

import jax
import jax.numpy as jnp
from jax import lax
from jax.experimental import pallas as pl
from jax.experimental.pallas import tpu as pltpu
from jax.experimental.pallas import tpu_sc as plsc

NUM_FEATS = 128
H = 200
W = 200
NC = 2
NS = 16
LANES = 16
NWORKERS = NC * NS
C_TOTAL = 2 * NUM_FEATS


def _make_kernel(bs):
    mesh = plsc.VectorSubcoreMesh(
        core_axis_name="c", subcore_axis_name="s",
        num_cores=NC, num_subcores=NS)

    n_half = NUM_FEATS // LANES
    y_groups = NWORKERS // bs
    slabs_per_w = H // y_groups
    n_pairs = slabs_per_w // 2

    @functools.partial(
        pl.kernel,
        out_type=jax.ShapeDtypeStruct((bs, H, W, C_TOTAL), jnp.float32),
        mesh=mesh,
        scratch_types=[
            pltpu.VMEM((H, NUM_FEATS), jnp.float32),
            pltpu.VMEM((W, C_TOTAL), jnp.float32),
            pltpu.VMEM((W, C_TOTAL), jnp.float32),
            pltpu.SemaphoreType.DMA,
            pltpu.SemaphoreType.DMA,
        ],
        compiler_params=pltpu.CompilerParams(use_tc_tiling_on_sc=True),
    )
    def body(rw_hbm, cw_hbm, out_hbm, rv, buf0, buf1, sem0, sem1):
        wid = lax.axis_index("s") * NC + lax.axis_index("c")
        bb = wid % bs
        y0 = wid // bs

        pltpu.sync_copy(rw_hbm, rv)
        h_cw0 = pltpu.async_copy(cw_hbm, buf0.at[:, pl.ds(0, NUM_FEATS)],
                                 sem0)
        h_cw1 = pltpu.async_copy(cw_hbm, buf1.at[:, pl.ds(0, NUM_FEATS)],
                                 sem1)

        def build(y, buf):
            vs = [rv[y, pl.ds(j * LANES, LANES)] for j in range(n_half)]

            def row_body(x, carry):
                for j in range(n_half):
                    buf[x, pl.ds(NUM_FEATS + j * LANES, LANES)] = vs[j]
                return carry

            lax.fori_loop(0, W, row_body, 0)

        def fire(y, buf, sem):
            pltpu.async_copy(buf, out_hbm.at[bb, y], sem)

        def drain(buf, sem):
            pltpu.make_async_copy(buf, out_hbm.at[bb, y0], sem).wait()

        build(y0, buf0)
        h_cw0.wait()
        fire(y0, buf0, sem0)
        build(y0 + y_groups, buf1)
        h_cw1.wait()
        fire(y0 + y_groups, buf1, sem1)

        def pair_body(i, carry):
            y_a = y0 + y_groups * (2 * i)
            drain(buf0, sem0)
            build(y_a, buf0)
            fire(y_a, buf0, sem0)
            y_b = y_a + y_groups
            drain(buf1, sem1)
            build(y_b, buf1)
            fire(y_b, buf1, sem1)
            return carry

        lax.fori_loop(1, n_pairs, pair_body, 0)

        drain(buf0, sem0)
        drain(buf1, sem1)

    return body


def kernel(mask, row_weight, col_weight):
    bs = mask.shape[0]
    out = _make_kernel(bs)(row_weight, col_weight)
    return jnp.transpose(out, (0, 3, 1, 2))


# speedup vs baseline: 1.0654x; 1.0654x over previous
import functools

import jax
import jax.numpy as jnp
from jax import lax
from jax.experimental import pallas as pl
from jax.experimental.pallas import tpu as pltpu
from jax.experimental.pallas import tpu_sc as plsc

NUM_FEATS = 128
H = 200
W = 200
NC = 2
NS = 16
LANES = 16
NWORKERS = NC * NS
C_TOTAL = 2 * NUM_FEATS


def _make_kernel(bs):
    mesh = plsc.VectorSubcoreMesh(
        core_axis_name="c", subcore_axis_name="s",
        num_cores=NC, num_subcores=NS)

    n_half = NUM_FEATS // LANES
    y_groups = NWORKERS // bs
    slabs_per_w = H // y_groups
    n_pairs = slabs_per_w // 2

    @functools.partial(
        pl.kernel,
        out_type=jax.ShapeDtypeStruct((bs, H, W, C_TOTAL), jnp.float32),
        mesh=mesh,
        scratch_types=[
            pltpu.VMEM((H, NUM_FEATS), jnp.float32),
            pltpu.VMEM((W, C_TOTAL), jnp.float32),
            pltpu.VMEM((W, C_TOTAL), jnp.float32),
            pltpu.SemaphoreType.DMA,
            pltpu.SemaphoreType.DMA,
        ],
        compiler_params=pltpu.CompilerParams(use_tc_tiling_on_sc=True),
    )
    def body(rw_hbm, cw_hbm, out_hbm, rv, buf0, buf1, sem0, sem1):
        wid = lax.axis_index("s") * NC + lax.axis_index("c")
        bb = wid % bs
        y0 = wid // bs

        pltpu.sync_copy(rw_hbm, rv)
        pltpu.sync_copy(cw_hbm, buf0.at[:, pl.ds(0, NUM_FEATS)])
        pltpu.sync_copy(cw_hbm, buf1.at[:, pl.ds(0, NUM_FEATS)])

        def build(y, buf):
            vs = [rv[y, pl.ds(j * LANES, LANES)] for j in range(n_half)]

            def row_body(x, carry):
                for j in range(n_half):
                    buf[x, pl.ds(NUM_FEATS + j * LANES, LANES)] = vs[j]
                return carry

            lax.fori_loop(0, W, row_body, 0)

        def fire(y, buf, sem):
            pltpu.async_copy(buf, out_hbm.at[bb, y], sem)

        def drain(buf, sem):
            pltpu.make_async_copy(buf, out_hbm.at[bb, y0], sem).wait()

        build(y0, buf0)
        fire(y0, buf0, sem0)
        build(y0 + y_groups, buf1)
        fire(y0 + y_groups, buf1, sem1)

        def pair_body(i, carry):
            y_a = y0 + y_groups * (2 * i)
            drain(buf0, sem0)
            build(y_a, buf0)
            fire(y_a, buf0, sem0)
            y_b = y_a + y_groups
            drain(buf1, sem1)
            build(y_b, buf1)
            fire(y_b, buf1, sem1)
            return carry

        lax.fori_loop(1, n_pairs, pair_body, 0)

        drain(buf0, sem0)
        drain(buf1, sem1)

    return body


def kernel(mask, row_weight, col_weight):
    bs = mask.shape[0]
    out = _make_kernel(bs)(row_weight, col_weight)
    return jnp.transpose(out, (0, 3, 1, 2))
